# baseline (speedup 1.0000x reference)
import functools

import jax
import jax.numpy as jnp
from jax import lax
from jax.experimental import pallas as pl
from jax.experimental.pallas import tpu as pltpu
from jax.experimental.pallas import tpu_sc as plsc

EMB = 128
NC = 2
NS = 16
NW = NC * NS
S = 2
CH = 128
NBUF = 6
LAG = 3

_MESH = plsc.VectorSubcoreMesh(core_axis_name="c", subcore_axis_name="s")


def _run_gather_pipeline(tasks, bufs, gsems, wsems):
    gpend = [None] * NBUF
    wpend = [None] * NBUF
    views = [None] * NBUF

    def drain(j):
        p = j % NBUF
        gpend[p].wait()
        buf, out, off, ln = views[p]
        wpend[p] = pltpu.async_copy(buf, out.at[pl.ds(off, ln)], wsems[p])

    for j, (tbl, idx, out, off, ln) in enumerate(tasks):
        p = j % NBUF
        buf = bufs[p] if ln == CH else bufs[p].at[pl.ds(0, ln)]
        if wpend[p] is not None:
            wpend[p].wait()
        gpend[p] = pltpu.async_copy(tbl.at[idx], buf, gsems[p])
        views[p] = (buf, out, off, ln)
        if j >= LAG:
            drain(j - LAG)
    for j in range(len(tasks) - LAG, len(tasks)):
        drain(j)
    for p in range(NBUF):
        if wpend[p] is not None:
            wpend[p].wait()


def _make_sc_fused(B, BKs):
    bpw = B // NW
    npw = BKs // NW
    n_chunks = npw // CH

    @functools.partial(
        pl.kernel,
        mesh=_MESH,
        out_type=(
            [jax.ShapeDtypeStruct((B, EMB), jnp.float32)] * 4
            + [jax.ShapeDtypeStruct((BKs, EMB), jnp.float32)] * 2
        ),
        scratch_types=(
            [pltpu.VMEM((bpw,), jnp.int32),
             pltpu.VMEM((bpw,), jnp.int32),
             pltpu.VMEM((n_chunks, CH), jnp.int32)]
            + [pltpu.VMEM((CH, EMB), jnp.float32) for _ in range(NBUF)]
            + [pltpu.SemaphoreType.DMA for _ in range(2 * NBUF)]
        ),
    )
    def sc_fused(u1, v1, n3, Q_hbm, U_hbm, T_hbm, V_hbm,
                 q_out, ue_out, t_out, ve_out, tn_out, vn_out,
                 idxu, idxv, idxn, *bufs_and_sems):
        bufs = bufs_and_sems[:NBUF]
        gsems = bufs_and_sems[NBUF:2 * NBUF]
        wsems = bufs_and_sems[2 * NBUF:]
        cid = lax.axis_index("c")
        sid = lax.axis_index("s")
        wid = sid * NC + cid
        base = wid * bpw
        nbase = wid * npw

        pltpu.sync_copy(u1.at[pl.ds(base, bpw)], idxu)
        pltpu.sync_copy(v1.at[pl.ds(base, bpw)], idxv)
        pltpu.sync_copy(n3.at[wid], idxn)

        tasks = [
            (Q_hbm, idxu, q_out, base, bpw),
            (U_hbm, idxu, ue_out, base, bpw),
            (T_hbm, idxv, t_out, base, bpw),
            (V_hbm, idxv, ve_out, base, bpw),
        ]
        for j in range(n_chunks):
            tasks.append((T_hbm, idxn.at[j], tn_out, nbase + j * CH, CH))
            tasks.append((V_hbm, idxn.at[j], vn_out, nbase + j * CH, CH))
        _run_gather_pipeline(tasks, bufs, gsems, wsems)

    return sc_fused


def _make_sc_neg(BKs):
    npw = BKs // NW
    n_chunks = npw // CH

    @functools.partial(
        pl.kernel,
        mesh=_MESH,
        out_type=[jax.ShapeDtypeStruct((BKs, EMB), jnp.float32)] * 2,
        scratch_types=(
            [pltpu.VMEM((n_chunks, CH), jnp.int32)]
            + [pltpu.VMEM((CH, EMB), jnp.float32) for _ in range(NBUF)]
            + [pltpu.SemaphoreType.DMA for _ in range(2 * NBUF)]
        ),
    )
    def sc_neg(n3, T_hbm, V_hbm, tn_out, vn_out, idxn, *bufs_and_sems):
        bufs = bufs_and_sems[:NBUF]
        gsems = bufs_and_sems[NBUF:2 * NBUF]
        wsems = bufs_and_sems[2 * NBUF:]
        cid = lax.axis_index("c")
        sid = lax.axis_index("s")
        wid = sid * NC + cid
        nbase = wid * npw

        pltpu.sync_copy(n3.at[wid], idxn)

        tasks = []
        for j in range(n_chunks):
            tasks.append((T_hbm, idxn.at[j], tn_out, nbase + j * CH, CH))
            tasks.append((V_hbm, idxn.at[j], vn_out, nbase + j * CH, CH))
        _run_gather_pipeline(tasks, bufs, gsems, wsems)

    return sc_neg


def _tc_pos_body(q_ref, ue_ref, t_ref, ve_ref, w_ref, b_ref, pred_ref):
    q = q_ref[...]
    h = ue_ref[...] * ve_ref[...]
    hw = lax.dot_general(h, w_ref[...], (((1,), (1,)), ((), ())),
                         preferred_element_type=jnp.float32)
    s = (q * (hw + b_ref[...])) * h
    pred_ref[...] = jnp.sum(s * jax.nn.sigmoid(t_ref[...]), axis=1,
                            keepdims=True)


def _make_tc_pos(B, Bb=512):
    return pl.pallas_call(
        _tc_pos_body,
        grid=(B // Bb,),
        in_specs=[
            pl.BlockSpec((Bb, EMB), lambda i: (i, 0)),
            pl.BlockSpec((Bb, EMB), lambda i: (i, 0)),
            pl.BlockSpec((Bb, EMB), lambda i: (i, 0)),
            pl.BlockSpec((Bb, EMB), lambda i: (i, 0)),
            pl.BlockSpec((EMB, EMB), lambda i: (0, 0)),
            pl.BlockSpec((1, EMB), lambda i: (0, 0)),
        ],
        out_specs=pl.BlockSpec((Bb, 1), lambda i: (i, 0)),
        out_shape=jax.ShapeDtypeStruct((B, 1), jnp.float32),
    )


def _tc_neg_body(K, Bb, q_ref, ue_ref, vn_ref, tn_ref, w_ref, b_ref,
                 predn_ref):
    q = q_ref[...]
    ue = ue_ref[...]
    w = w_ref[...]
    b = b_ref[...]

    hn = vn_ref[...] * ue[None]
    hnw = lax.dot_general(hn.reshape(K * Bb, EMB), w, (((1,), (1,)), ((), ())),
                          preferred_element_type=jnp.float32)
    an = (hnw.reshape(K, Bb, EMB) + b[None]) * q[None]
    sn = an * hn
    predn_ref[...] = jnp.sum(sn * jax.nn.sigmoid(tn_ref[...]), axis=2,
                             keepdims=True)


def _make_tc_neg(B, Ks, Bb=128):
    return pl.pallas_call(
        functools.partial(_tc_neg_body, Ks, Bb),
        grid=(B // Bb,),
        in_specs=[
            pl.BlockSpec((Bb, EMB), lambda i: (i, 0)),
            pl.BlockSpec((Bb, EMB), lambda i: (i, 0)),
            pl.BlockSpec((Ks, Bb, EMB), lambda i: (0, i, 0)),
            pl.BlockSpec((Ks, Bb, EMB), lambda i: (0, i, 0)),
            pl.BlockSpec((EMB, EMB), lambda i: (0, 0)),
            pl.BlockSpec((1, EMB), lambda i: (0, 0)),
        ],
        out_specs=pl.BlockSpec((Ks, Bb, 1), lambda i: (0, i, 0)),
        out_shape=jax.ShapeDtypeStruct((Ks, B, 1), jnp.float32),
    )


def kernel(u, v, n, U_emb, Q_emb, V_emb, T_emb, W, b):
    B = u.shape[0]
    K = n.shape[1]
    KSPLIT = (12, 8)

    u1 = u.astype(jnp.int32)
    v1 = v.astype(jnp.int32)
    n32 = n.astype(jnp.int32)
    b2 = b.reshape(1, EMB)

    sc_fused = _make_sc_fused(B, B * KSPLIT[0])
    tc_pos = _make_tc_pos(B)

    def _nT(k0, k1):
        return n32[:, k0:k1].T.reshape(NW, B * (k1 - k0) // NW // CH, CH)

    q, ue, t, ve, tn0, vn0 = sc_fused(u1, v1, _nT(0, KSPLIT[0]),
                                      Q_emb, U_emb, T_emb, V_emb)
    negs = [(KSPLIT[0], tn0, vn0)]
    k0 = KSPLIT[0]
    for ks in KSPLIT[1:]:
        sc_neg = _make_sc_neg(B * ks)
        negs.append((ks,) + tuple(sc_neg(_nT(k0, k0 + ks), T_emb, V_emb)))
        k0 += ks

    pred = tc_pos(q, ue, t, ve, W, b2).reshape(B)

    predns = []
    for ks, tn, vn in negs:
        tc_neg = _make_tc_neg(B, ks)
        prednT_s = tc_neg(q, ue,
                          vn.reshape(ks, B, EMB), tn.reshape(ks, B, EMB),
                          W, b2)
        predns.append(prednT_s.reshape(ks, B).T)

    predn = jnp.concatenate(predns, axis=1).reshape(B * K)
    return (pred, predn)


# speedup vs baseline: 1.1454x; 1.1454x over previous
import functools

import jax
import jax.numpy as jnp
from jax import lax
from jax.experimental import pallas as pl
from jax.experimental.pallas import tpu as pltpu
from jax.experimental.pallas import tpu_sc as plsc

EMB = 128
NC = 2
NS = 16
NW = NC * NS
S = 2
CH = 128
NBUF = 4
LAG = 2

_MESH = plsc.VectorSubcoreMesh(core_axis_name="c", subcore_axis_name="s")


def _make_sc_gather(Bs, BKs):
    bpw = Bs // NW
    npw = BKs // NW
    n_chunks = npw // CH

    @functools.partial(
        pl.kernel,
        mesh=_MESH,
        out_type=[
            jax.ShapeDtypeStruct((Bs, EMB), jnp.float32),
            jax.ShapeDtypeStruct((Bs, EMB), jnp.float32),
            jax.ShapeDtypeStruct((Bs, EMB), jnp.float32),
            jax.ShapeDtypeStruct((Bs, EMB), jnp.float32),
            jax.ShapeDtypeStruct((BKs, EMB), jnp.float32),
            jax.ShapeDtypeStruct((BKs, EMB), jnp.float32),
        ],
        scratch_types=(
            [pltpu.VMEM((bpw,), jnp.int32),
             pltpu.VMEM((bpw,), jnp.int32),
             pltpu.VMEM((n_chunks, CH), jnp.int32)]
            + [pltpu.VMEM((CH, EMB), jnp.float32) for _ in range(NBUF)]
            + [pltpu.SemaphoreType.DMA for _ in range(2 * NBUF)]
        ),
    )
    def sc_gather(u1, v1, n3, Q_hbm, U_hbm, T_hbm, V_hbm,
                  q_out, ue_out, t_out, ve_out, tn_out, vn_out,
                  idxu, idxv, idxn, *bufs_and_sems):
        bufs = bufs_and_sems[:NBUF]
        gsems = bufs_and_sems[NBUF:2 * NBUF]
        wsems = bufs_and_sems[2 * NBUF:]
        cid = lax.axis_index("c")
        sid = lax.axis_index("s")
        wid = sid * NC + cid
        base = wid * bpw
        nbase = wid * npw

        pltpu.sync_copy(u1.at[pl.ds(base, bpw)], idxu)
        pltpu.sync_copy(v1.at[pl.ds(base, bpw)], idxv)
        pltpu.sync_copy(n3.at[wid], idxn)

        tasks = [
            (Q_hbm, idxu, q_out, base, bpw),
            (U_hbm, idxu, ue_out, base, bpw),
            (T_hbm, idxv, t_out, base, bpw),
            (V_hbm, idxv, ve_out, base, bpw),
        ]
        for j in range(n_chunks):
            tasks.append((T_hbm, idxn.at[j], tn_out, nbase + j * CH, CH))
            tasks.append((V_hbm, idxn.at[j], vn_out, nbase + j * CH, CH))

        gpend = [None] * NBUF
        wpend = [None] * NBUF
        views = [None] * NBUF

        def drain(j):
            p = j % NBUF
            gpend[p].wait()
            buf, out, off, ln = views[p]
            wpend[p] = pltpu.async_copy(buf, out.at[pl.ds(off, ln)], wsems[p])

        for j, (tbl, idx, out, off, ln) in enumerate(tasks):
            p = j % NBUF
            buf = bufs[p] if ln == CH else bufs[p].at[pl.ds(0, ln)]
            if wpend[p] is not None:
                wpend[p].wait()
            gpend[p] = pltpu.async_copy(tbl.at[idx], buf, gsems[p])
            views[p] = (buf, out, off, ln)
            if j >= LAG:
                drain(j - LAG)
        for j in range(len(tasks) - LAG, len(tasks)):
            drain(j)
        for p in range(NBUF):
            if wpend[p] is not None:
                wpend[p].wait()

    return sc_gather


def _tc_body(K, Bb, q_ref, ue_ref, t_ref, ve_ref, vn_ref, tn_ref, w_ref, b_ref,
             pred_ref, predn_ref):
    q = q_ref[...]
    ue = ue_ref[...]
    w = w_ref[...]
    b = b_ref[...]

    h = ue * ve_ref[...]
    hw = lax.dot_general(h, w, (((1,), (1,)), ((), ())),
                         preferred_element_type=jnp.float32)
    s = (q * (hw + b)) * h
    pred_ref[...] = jnp.sum(s * jax.nn.sigmoid(t_ref[...]), axis=1,
                            keepdims=True)

    hn = vn_ref[...] * ue[None]
    hnw = lax.dot_general(hn.reshape(K * Bb, EMB), w, (((1,), (1,)), ((), ())),
                          preferred_element_type=jnp.float32)
    an = (hnw.reshape(K, Bb, EMB) + b[None]) * q[None]
    sn = an * hn
    predn_ref[...] = jnp.sum(sn * jax.nn.sigmoid(tn_ref[...]), axis=2,
                             keepdims=True)


def _make_tc_compute(Bs, K, Bb=128):
    nb = Bs // Bb

    return pl.pallas_call(
        functools.partial(_tc_body, K, Bb),
        grid=(nb,),
        in_specs=[
            pl.BlockSpec((Bb, EMB), lambda i: (i, 0)),
            pl.BlockSpec((Bb, EMB), lambda i: (i, 0)),
            pl.BlockSpec((Bb, EMB), lambda i: (i, 0)),
            pl.BlockSpec((Bb, EMB), lambda i: (i, 0)),
            pl.BlockSpec((K, Bb, EMB), lambda i: (0, i, 0)),
            pl.BlockSpec((K, Bb, EMB), lambda i: (0, i, 0)),
            pl.BlockSpec((EMB, EMB), lambda i: (0, 0)),
            pl.BlockSpec((1, EMB), lambda i: (0, 0)),
        ],
        out_specs=[
            pl.BlockSpec((Bb, 1), lambda i: (i, 0)),
            pl.BlockSpec((K, Bb, 1), lambda i: (0, i, 0)),
        ],
        out_shape=[
            jax.ShapeDtypeStruct((Bs, 1), jnp.float32),
            jax.ShapeDtypeStruct((K, Bs, 1), jnp.float32),
        ],
    )


def kernel(u, v, n, U_emb, Q_emb, V_emb, T_emb, W, b):
    B = u.shape[0]
    K = n.shape[1]
    Bs = B // S
    BKs = Bs * K

    u1 = u.astype(jnp.int32)
    v1 = v.astype(jnp.int32)
    n32 = n.astype(jnp.int32)
    b2 = b.reshape(1, EMB)

    sc_gather = _make_sc_gather(Bs, BKs)
    tc = _make_tc_compute(Bs, K)

    gathered = []
    for s in range(S):
        sl = slice(s * Bs, (s + 1) * Bs)
        nT = n32[sl].T.reshape(NW, BKs // NW // CH, CH)
        gathered.append(sc_gather(u1[sl], v1[sl], nT,
                                  Q_emb, U_emb, T_emb, V_emb))

    preds = []
    predns = []
    for s in range(S):
        q, ue, t, ve, tn, vn = gathered[s]
        pred_s, prednT_s = tc(q, ue, t, ve,
                              vn.reshape(K, Bs, EMB), tn.reshape(K, Bs, EMB),
                              W, b2)
        preds.append(pred_s.reshape(Bs))
        predns.append(prednT_s.reshape(K, Bs).T)

    pred = jnp.concatenate(preds)
    predn = jnp.concatenate(predns).reshape(B * K)
    return (pred, predn)


# speedup vs baseline: 1.1465x; 1.0009x over previous
import functools

import jax
import jax.numpy as jnp
from jax import lax
from jax.experimental import pallas as pl
from jax.experimental.pallas import tpu as pltpu
from jax.experimental.pallas import tpu_sc as plsc

EMB = 128
NC = 2
NS = 16
NW = NC * NS
S = 2
CH = 128
NBUF = 4
LAG = 2

_MESH = plsc.VectorSubcoreMesh(core_axis_name="c", subcore_axis_name="s")


def _make_sc_gather(Bs, BKs):
    bpw = Bs // NW
    npw = BKs // NW
    n_chunks = npw // CH

    @functools.partial(
        pl.kernel,
        mesh=_MESH,
        out_type=[
            jax.ShapeDtypeStruct((Bs, EMB), jnp.float32),
            jax.ShapeDtypeStruct((Bs, EMB), jnp.float32),
            jax.ShapeDtypeStruct((Bs, EMB), jnp.float32),
            jax.ShapeDtypeStruct((Bs, EMB), jnp.float32),
            jax.ShapeDtypeStruct((BKs, EMB), jnp.float32),
            jax.ShapeDtypeStruct((BKs, EMB), jnp.float32),
        ],
        scratch_types=(
            [pltpu.VMEM((bpw,), jnp.int32),
             pltpu.VMEM((bpw,), jnp.int32),
             pltpu.VMEM((n_chunks, CH), jnp.int32)]
            + [pltpu.VMEM((CH, EMB), jnp.float32) for _ in range(NBUF)]
            + [pltpu.SemaphoreType.DMA for _ in range(2 * NBUF)]
        ),
    )
    def sc_gather(u1, v1, n3, Q_hbm, U_hbm, T_hbm, V_hbm,
                  q_out, ue_out, t_out, ve_out, tn_out, vn_out,
                  idxu, idxv, idxn, *bufs_and_sems):
        bufs = bufs_and_sems[:NBUF]
        gsems = bufs_and_sems[NBUF:2 * NBUF]
        wsems = bufs_and_sems[2 * NBUF:]
        cid = lax.axis_index("c")
        sid = lax.axis_index("s")
        wid = sid * NC + cid
        base = wid * bpw
        nbase = wid * npw

        pltpu.sync_copy(u1.at[pl.ds(base, bpw)], idxu)
        pltpu.sync_copy(v1.at[pl.ds(base, bpw)], idxv)
        pltpu.sync_copy(n3.at[wid], idxn)

        tasks = [
            (Q_hbm, idxu, q_out, base, bpw),
            (U_hbm, idxu, ue_out, base, bpw),
            (T_hbm, idxv, t_out, base, bpw),
            (V_hbm, idxv, ve_out, base, bpw),
        ]
        for j in range(n_chunks):
            tasks.append((T_hbm, idxn.at[j], tn_out, nbase + j * CH, CH))
            tasks.append((V_hbm, idxn.at[j], vn_out, nbase + j * CH, CH))

        gpend = [None] * NBUF
        wpend = [None] * NBUF
        views = [None] * NBUF

        def drain(j):
            p = j % NBUF
            gpend[p].wait()
            buf, out, off, ln = views[p]
            wpend[p] = pltpu.async_copy(buf, out.at[pl.ds(off, ln)], wsems[p])

        for j, (tbl, idx, out, off, ln) in enumerate(tasks):
            p = j % NBUF
            buf = bufs[p] if ln == CH else bufs[p].at[pl.ds(0, ln)]
            if wpend[p] is not None:
                wpend[p].wait()
            gpend[p] = pltpu.async_copy(tbl.at[idx], buf, gsems[p])
            views[p] = (buf, out, off, ln)
            if j >= LAG:
                drain(j - LAG)
        for j in range(len(tasks) - LAG, len(tasks)):
            drain(j)
        for p in range(NBUF):
            if wpend[p] is not None:
                wpend[p].wait()

    return sc_gather


def _tc_body(K, Bb, q_ref, ue_ref, t_ref, ve_ref, vn_ref, tn_ref, w_ref, b_ref,
             pred_ref, predn_ref):
    q = q_ref[...]
    ue = ue_ref[...]
    w = w_ref[...]
    b = b_ref[...]

    h = ue * ve_ref[...]
    hw = lax.dot_general(h, w, (((1,), (1,)), ((), ())),
                         preferred_element_type=jnp.float32)
    s = (q * (hw + b)) * h
    pred_ref[...] = jnp.sum(s * jax.nn.sigmoid(t_ref[...]), axis=1,
                            keepdims=True)

    hn = vn_ref[...] * ue[None]
    hnw = lax.dot_general(hn.reshape(K * Bb, EMB), w, (((1,), (1,)), ((), ())),
                          preferred_element_type=jnp.float32)
    an = (hnw.reshape(K, Bb, EMB) + b[None]) * q[None]
    sn = an * hn
    predn_ref[...] = jnp.sum(sn * jax.nn.sigmoid(tn_ref[...]), axis=2,
                             keepdims=True)


def _make_tc_compute(Bs, K, Bb=128):
    nb = Bs // Bb

    return pl.pallas_call(
        functools.partial(_tc_body, K, Bb),
        grid=(nb,),
        in_specs=[
            pl.BlockSpec((Bb, EMB), lambda i: (i, 0)),
            pl.BlockSpec((Bb, EMB), lambda i: (i, 0)),
            pl.BlockSpec((Bb, EMB), lambda i: (i, 0)),
            pl.BlockSpec((Bb, EMB), lambda i: (i, 0)),
            pl.BlockSpec((K, Bb, EMB), lambda i: (0, i, 0)),
            pl.BlockSpec((K, Bb, EMB), lambda i: (0, i, 0)),
            pl.BlockSpec((EMB, EMB), lambda i: (0, 0)),
            pl.BlockSpec((1, EMB), lambda i: (0, 0)),
        ],
        out_specs=[
            pl.BlockSpec((Bb, 1), lambda i: (i, 0)),
            pl.BlockSpec((K, Bb, 1), lambda i: (0, i, 0)),
        ],
        out_shape=[
            jax.ShapeDtypeStruct((Bs, 1), jnp.float32),
            jax.ShapeDtypeStruct((K, Bs, 1), jnp.float32),
        ],
    )


def kernel(u, v, n, U_emb, Q_emb, V_emb, T_emb, W, b):
    B = u.shape[0]
    K = n.shape[1]
    Bs = B // S
    BKs = Bs * K

    u1 = u.astype(jnp.int32)
    v1 = v.astype(jnp.int32)
    n32 = n.astype(jnp.int32)
    b2 = b.reshape(1, EMB)

    sc_gather = _make_sc_gather(Bs, BKs)
    tc = _make_tc_compute(Bs, K)

    preds = []
    predns = []
    for s in range(S):
        sl = slice(s * Bs, (s + 1) * Bs)
        nT = n32[sl].T.reshape(NW, BKs // NW // CH, CH)
        q, ue, t, ve, tn, vn = sc_gather(u1[sl], v1[sl], nT,
                                         Q_emb, U_emb, T_emb, V_emb)
        pred_s, prednT_s = tc(q, ue, t, ve,
                              vn.reshape(K, Bs, EMB), tn.reshape(K, Bs, EMB),
                              W, b2)
        preds.append(pred_s.reshape(Bs))
        predns.append(prednT_s.reshape(K, Bs).T)

    pred = jnp.concatenate(preds)
    predn = jnp.concatenate(predns).reshape(B * K)
    return (pred, predn)


# speedup vs baseline: 1.1848x; 1.0334x over previous
import functools

import jax
import jax.numpy as jnp
from jax import lax
from jax.experimental import pallas as pl
from jax.experimental.pallas import tpu as pltpu
from jax.experimental.pallas import tpu_sc as plsc

EMB = 128
NC = 2
NS = 16
NW = NC * NS
S = 2
CH = 128
NBUF = 4
LAG = 2

_MESH = plsc.VectorSubcoreMesh(core_axis_name="c", subcore_axis_name="s")


def _make_sc_gather(Bs, BKs):
    bpw = Bs // NW
    npw = BKs // NW
    n_chunks = npw // CH

    @functools.partial(
        pl.kernel,
        mesh=_MESH,
        out_type=[
            jax.ShapeDtypeStruct((Bs, EMB), jnp.float32),
            jax.ShapeDtypeStruct((Bs, EMB), jnp.float32),
            jax.ShapeDtypeStruct((Bs, EMB), jnp.float32),
            jax.ShapeDtypeStruct((Bs, EMB), jnp.float32),
            jax.ShapeDtypeStruct((BKs, EMB), jnp.float32),
            jax.ShapeDtypeStruct((BKs, EMB), jnp.float32),
        ],
        scratch_types=(
            [pltpu.VMEM((bpw,), jnp.int32),
             pltpu.VMEM((bpw,), jnp.int32),
             pltpu.VMEM((n_chunks, CH), jnp.int32)]
            + [pltpu.VMEM((CH, EMB), jnp.float32) for _ in range(NBUF)]
            + [pltpu.SemaphoreType.DMA for _ in range(2 * NBUF)]
        ),
    )
    def sc_gather(u1, v1, n3, Q_hbm, U_hbm, T_hbm, V_hbm,
                  q_out, ue_out, t_out, ve_out, tn_out, vn_out,
                  idxu, idxv, idxn, *bufs_and_sems):
        bufs = bufs_and_sems[:NBUF]
        gsems = bufs_and_sems[NBUF:2 * NBUF]
        wsems = bufs_and_sems[2 * NBUF:]
        cid = lax.axis_index("c")
        sid = lax.axis_index("s")
        wid = sid * NC + cid
        base = wid * bpw
        nbase = wid * npw

        pltpu.sync_copy(u1.at[pl.ds(base, bpw)], idxu)
        pltpu.sync_copy(v1.at[pl.ds(base, bpw)], idxv)
        pltpu.sync_copy(n3.at[wid], idxn)

        tasks = [
            (Q_hbm, idxu, q_out, base, bpw),
            (U_hbm, idxu, ue_out, base, bpw),
            (T_hbm, idxv, t_out, base, bpw),
            (V_hbm, idxv, ve_out, base, bpw),
        ]
        for j in range(n_chunks):
            tasks.append((T_hbm, idxn.at[j], tn_out, nbase + j * CH, CH))
            tasks.append((V_hbm, idxn.at[j], vn_out, nbase + j * CH, CH))

        gpend = [None] * NBUF
        wpend = [None] * NBUF
        views = [None] * NBUF

        def drain(j):
            p = j % NBUF
            gpend[p].wait()
            buf, out, off, ln = views[p]
            wpend[p] = pltpu.async_copy(buf, out.at[pl.ds(off, ln)], wsems[p])

        for j, (tbl, idx, out, off, ln) in enumerate(tasks):
            p = j % NBUF
            buf = bufs[p] if ln == CH else bufs[p].at[pl.ds(0, ln)]
            if wpend[p] is not None:
                wpend[p].wait()
            gpend[p] = pltpu.async_copy(tbl.at[idx], buf, gsems[p])
            views[p] = (buf, out, off, ln)
            if j >= LAG:
                drain(j - LAG)
        for j in range(len(tasks) - LAG, len(tasks)):
            drain(j)
        for p in range(NBUF):
            if wpend[p] is not None:
                wpend[p].wait()

    return sc_gather


def _tc_body(K, Bb, q_ref, ue_ref, t_ref, ve_ref, vn_ref, tn_ref, w_ref, b_ref,
             pred_ref, predn_ref):
    q = q_ref[...]
    ue = ue_ref[...]
    w = w_ref[...]
    b = b_ref[...]

    h = ue * ve_ref[...]
    hw = lax.dot_general(h, w, (((1,), (1,)), ((), ())),
                         preferred_element_type=jnp.float32)
    s = (q * (hw + b)) * h
    pred_ref[...] = jnp.sum(s * jax.nn.sigmoid(t_ref[...]), axis=1,
                            keepdims=True)

    hn = vn_ref[...] * ue[None]
    hnw = lax.dot_general(hn.reshape(K * Bb, EMB), w, (((1,), (1,)), ((), ())),
                          preferred_element_type=jnp.float32)
    an = (hnw.reshape(K, Bb, EMB) + b[None]) * q[None]
    sn = an * hn
    predn_ref[...] = jnp.sum(sn * jax.nn.sigmoid(tn_ref[...]), axis=2,
                             keepdims=True)


def _make_tc_compute(Bs, K, Bb=256):
    nb = Bs // Bb

    return pl.pallas_call(
        functools.partial(_tc_body, K, Bb),
        grid=(nb,),
        in_specs=[
            pl.BlockSpec((Bb, EMB), lambda i: (i, 0)),
            pl.BlockSpec((Bb, EMB), lambda i: (i, 0)),
            pl.BlockSpec((Bb, EMB), lambda i: (i, 0)),
            pl.BlockSpec((Bb, EMB), lambda i: (i, 0)),
            pl.BlockSpec((K, Bb, EMB), lambda i: (0, i, 0)),
            pl.BlockSpec((K, Bb, EMB), lambda i: (0, i, 0)),
            pl.BlockSpec((EMB, EMB), lambda i: (0, 0)),
            pl.BlockSpec((1, EMB), lambda i: (0, 0)),
        ],
        out_specs=[
            pl.BlockSpec((Bb, 1), lambda i: (i, 0)),
            pl.BlockSpec((K, Bb, 1), lambda i: (0, i, 0)),
        ],
        out_shape=[
            jax.ShapeDtypeStruct((Bs, 1), jnp.float32),
            jax.ShapeDtypeStruct((K, Bs, 1), jnp.float32),
        ],
    )


def kernel(u, v, n, U_emb, Q_emb, V_emb, T_emb, W, b):
    B = u.shape[0]
    K = n.shape[1]
    Bs = B // S
    BKs = Bs * K

    u1 = u.astype(jnp.int32)
    v1 = v.astype(jnp.int32)
    n32 = n.astype(jnp.int32)
    b2 = b.reshape(1, EMB)

    sc_gather = _make_sc_gather(Bs, BKs)
    tc = _make_tc_compute(Bs, K)

    preds = []
    predns = []
    for s in range(S):
        sl = slice(s * Bs, (s + 1) * Bs)
        nT = n32[sl].T.reshape(NW, BKs // NW // CH, CH)
        q, ue, t, ve, tn, vn = sc_gather(u1[sl], v1[sl], nT,
                                         Q_emb, U_emb, T_emb, V_emb)
        pred_s, prednT_s = tc(q, ue, t, ve,
                              vn.reshape(K, Bs, EMB), tn.reshape(K, Bs, EMB),
                              W, b2)
        preds.append(pred_s.reshape(Bs))
        predns.append(prednT_s.reshape(K, Bs).T)

    pred = jnp.concatenate(preds)
    predn = jnp.concatenate(predns).reshape(B * K)
    return (pred, predn)


# speedup vs baseline: 1.1989x; 1.0119x over previous
import functools

import jax
import jax.numpy as jnp
from jax import lax
from jax.experimental import pallas as pl
from jax.experimental.pallas import tpu as pltpu
from jax.experimental.pallas import tpu_sc as plsc

EMB = 128
NC = 2
NS = 16
NW = NC * NS
S = 2
CH = 128
NBUF = 4
LAG = 2

_MESH = plsc.VectorSubcoreMesh(core_axis_name="c", subcore_axis_name="s")


def _make_sc_gather(Bs, BKs):
    bpw = Bs // NW
    npw = BKs // NW
    n_chunks = npw // CH

    @functools.partial(
        pl.kernel,
        mesh=_MESH,
        out_type=[
            jax.ShapeDtypeStruct((Bs, EMB), jnp.float32),
            jax.ShapeDtypeStruct((Bs, EMB), jnp.float32),
            jax.ShapeDtypeStruct((Bs, EMB), jnp.float32),
            jax.ShapeDtypeStruct((Bs, EMB), jnp.float32),
            jax.ShapeDtypeStruct((BKs, EMB), jnp.float32),
            jax.ShapeDtypeStruct((BKs, EMB), jnp.float32),
        ],
        scratch_types=(
            [pltpu.VMEM((bpw,), jnp.int32),
             pltpu.VMEM((bpw,), jnp.int32),
             pltpu.VMEM((n_chunks, CH), jnp.int32)]
            + [pltpu.VMEM((CH, EMB), jnp.float32) for _ in range(NBUF)]
            + [pltpu.SemaphoreType.DMA for _ in range(2 * NBUF)]
        ),
    )
    def sc_gather(u1, v1, n3, Q_hbm, U_hbm, T_hbm, V_hbm,
                  q_out, ue_out, t_out, ve_out, tn_out, vn_out,
                  idxu, idxv, idxn, *bufs_and_sems):
        bufs = bufs_and_sems[:NBUF]
        gsems = bufs_and_sems[NBUF:2 * NBUF]
        wsems = bufs_and_sems[2 * NBUF:]
        cid = lax.axis_index("c")
        sid = lax.axis_index("s")
        wid = sid * NC + cid
        base = wid * bpw
        nbase = wid * npw

        pltpu.sync_copy(u1.at[pl.ds(base, bpw)], idxu)
        pltpu.sync_copy(v1.at[pl.ds(base, bpw)], idxv)
        pltpu.sync_copy(n3.at[wid], idxn)

        tasks = [
            (Q_hbm, idxu, q_out, base, bpw),
            (U_hbm, idxu, ue_out, base, bpw),
            (T_hbm, idxv, t_out, base, bpw),
            (V_hbm, idxv, ve_out, base, bpw),
        ]
        for j in range(n_chunks):
            tasks.append((T_hbm, idxn.at[j], tn_out, nbase + j * CH, CH))
            tasks.append((V_hbm, idxn.at[j], vn_out, nbase + j * CH, CH))

        gpend = [None] * NBUF
        wpend = [None] * NBUF
        views = [None] * NBUF

        def drain(j):
            p = j % NBUF
            gpend[p].wait()
            buf, out, off, ln = views[p]
            wpend[p] = pltpu.async_copy(buf, out.at[pl.ds(off, ln)], wsems[p])

        for j, (tbl, idx, out, off, ln) in enumerate(tasks):
            p = j % NBUF
            buf = bufs[p] if ln == CH else bufs[p].at[pl.ds(0, ln)]
            if wpend[p] is not None:
                wpend[p].wait()
            gpend[p] = pltpu.async_copy(tbl.at[idx], buf, gsems[p])
            views[p] = (buf, out, off, ln)
            if j >= LAG:
                drain(j - LAG)
        for j in range(len(tasks) - LAG, len(tasks)):
            drain(j)
        for p in range(NBUF):
            if wpend[p] is not None:
                wpend[p].wait()

    return sc_gather


def _tc_body(K, Bb, q_ref, ue_ref, t_ref, ve_ref, vn_ref, tn_ref, w_ref, b_ref,
             pred_ref, predn_ref):
    q = q_ref[...]
    ue = ue_ref[...]
    w = w_ref[...]
    b = b_ref[...]

    h = ue * ve_ref[...]
    hw = lax.dot_general(h, w, (((1,), (1,)), ((), ())),
                         preferred_element_type=jnp.float32)
    s = (q * (hw + b)) * h
    pred_ref[...] = jnp.sum(s * jax.nn.sigmoid(t_ref[...]), axis=1,
                            keepdims=True)

    hn = vn_ref[...] * ue[None]
    hnw = lax.dot_general(hn.reshape(K * Bb, EMB), w, (((1,), (1,)), ((), ())),
                          preferred_element_type=jnp.float32)
    an = (hnw.reshape(K, Bb, EMB) + b[None]) * q[None]
    sn = an * hn
    predn_ref[...] = jnp.sum(sn * jax.nn.sigmoid(tn_ref[...]), axis=2,
                             keepdims=True)


def _make_tc_compute(Bs, K, Bb=512):
    nb = Bs // Bb

    return pl.pallas_call(
        functools.partial(_tc_body, K, Bb),
        grid=(nb,),
        in_specs=[
            pl.BlockSpec((Bb, EMB), lambda i: (i, 0)),
            pl.BlockSpec((Bb, EMB), lambda i: (i, 0)),
            pl.BlockSpec((Bb, EMB), lambda i: (i, 0)),
            pl.BlockSpec((Bb, EMB), lambda i: (i, 0)),
            pl.BlockSpec((K, Bb, EMB), lambda i: (0, i, 0)),
            pl.BlockSpec((K, Bb, EMB), lambda i: (0, i, 0)),
            pl.BlockSpec((EMB, EMB), lambda i: (0, 0)),
            pl.BlockSpec((1, EMB), lambda i: (0, 0)),
        ],
        out_specs=[
            pl.BlockSpec((Bb, 1), lambda i: (i, 0)),
            pl.BlockSpec((K, Bb, 1), lambda i: (0, i, 0)),
        ],
        out_shape=[
            jax.ShapeDtypeStruct((Bs, 1), jnp.float32),
            jax.ShapeDtypeStruct((K, Bs, 1), jnp.float32),
        ],
    )


def kernel(u, v, n, U_emb, Q_emb, V_emb, T_emb, W, b):
    B = u.shape[0]
    K = n.shape[1]
    Bs = B // S
    BKs = Bs * K

    u1 = u.astype(jnp.int32)
    v1 = v.astype(jnp.int32)
    n32 = n.astype(jnp.int32)
    b2 = b.reshape(1, EMB)

    sc_gather = _make_sc_gather(Bs, BKs)
    tc = _make_tc_compute(Bs, K)

    preds = []
    predns = []
    for s in range(S):
        sl = slice(s * Bs, (s + 1) * Bs)
        nT = n32[sl].T.reshape(NW, BKs // NW // CH, CH)
        q, ue, t, ve, tn, vn = sc_gather(u1[sl], v1[sl], nT,
                                         Q_emb, U_emb, T_emb, V_emb)
        pred_s, prednT_s = tc(q, ue, t, ve,
                              vn.reshape(K, Bs, EMB), tn.reshape(K, Bs, EMB),
                              W, b2)
        preds.append(pred_s.reshape(Bs))
        predns.append(prednT_s.reshape(K, Bs).T)

    pred = jnp.concatenate(preds)
    predn = jnp.concatenate(predns).reshape(B * K)
    return (pred, predn)
